# baseline (speedup 1.0000x reference)
import jax
from jax.experimental import pallas as pl
from jax.experimental.pallas import tpu as pltpu

_ROWS = 100000
_COLS = 128
_N_CHUNKS = 10
_CHUNK = _ROWS // _N_CHUNKS
_N_BUF = 3


def _identity_copy_kernel(x_ref, o_ref, b0, b1, b2, isem, osem):
    bufs = (b0, b1, b2)

    def in_copy(i, b):
        return pltpu.make_async_copy(
            x_ref.at[pl.ds(i * _CHUNK, _CHUNK), :], bufs[b], isem.at[b])

    def out_copy(i, b):
        return pltpu.make_async_copy(
            bufs[b], o_ref.at[pl.ds(i * _CHUNK, _CHUNK), :], osem.at[b])

    for b in range(_N_BUF):
        in_copy(b, b).start()
    for i in range(_N_CHUNKS):
        b = i % _N_BUF
        in_copy(i, b).wait()
        out_copy(i, b).start()
        if i + _N_BUF < _N_CHUNKS:
            out_copy(i, b).wait()
            in_copy(i + _N_BUF, b).start()
    for i in range(_N_CHUNKS - _N_BUF, _N_CHUNKS):
        out_copy(i, i % _N_BUF).wait()


def kernel(x):
    return pl.pallas_call(
        _identity_copy_kernel,
        in_specs=[pl.BlockSpec(memory_space=pl.ANY)],
        out_specs=pl.BlockSpec(memory_space=pl.ANY),
        out_shape=jax.ShapeDtypeStruct(x.shape, x.dtype),
        scratch_shapes=[
            pltpu.VMEM((_CHUNK, _COLS), x.dtype),
            pltpu.VMEM((_CHUNK, _COLS), x.dtype),
            pltpu.VMEM((_CHUNK, _COLS), x.dtype),
            pltpu.SemaphoreType.DMA((_N_BUF,)),
            pltpu.SemaphoreType.DMA((_N_BUF,)),
        ],
    )(x)


# speedup vs baseline: 1.0640x; 1.0640x over previous
import jax
from jax.experimental import pallas as pl
from jax.experimental.pallas import tpu as pltpu


_BLOCK_ROWS = 18400


def _identity_copy_kernel(x_ref, o_ref):
    o_ref[...] = x_ref[...]


def kernel(x):
    rows = x.shape[0]
    return pl.pallas_call(
        _identity_copy_kernel,
        grid=(pl.cdiv(rows, _BLOCK_ROWS),),
        in_specs=[pl.BlockSpec((_BLOCK_ROWS, x.shape[1]), lambda i: (i, 0))],
        out_specs=pl.BlockSpec((_BLOCK_ROWS, x.shape[1]), lambda i: (i, 0)),
        out_shape=jax.ShapeDtypeStruct(x.shape, x.dtype),
        compiler_params=pltpu.CompilerParams(
            dimension_semantics=("parallel",),
        ),
    )(x)
